# baseline (speedup 1.0000x reference)
import functools

import jax
import jax.numpy as jnp
from jax import lax
from jax.experimental import pallas as pl
from jax.experimental.pallas import tpu as pltpu
from jax.experimental.pallas import tpu_sc as plsc

_ROWS = 4 * 8192
_N = 2048
_R = 512
_G = _ROWS // _R
_LSV = 16


def _sc_base(ra_flat, lcm_flat):
    info = plsc.get_sparse_core_info()
    nw = info.num_cores * info.num_subcores
    cols = _N // nw
    mesh = plsc.VectorSubcoreMesh(core_axis_name="c", subcore_axis_name="s")

    @functools.partial(
        pl.kernel,
        mesh=mesh,
        out_type=jax.ShapeDtypeStruct((_N,), jnp.float32),
        scratch_types=[
            pltpu.VMEM((_LSV,), jnp.float32),
            pltpu.VMEM((_LSV, cols), jnp.float32),
            pltpu.VMEM((cols,), jnp.float32),
            pltpu.SemaphoreType.DMA,
        ],
    )
    def body(ra_hbm, lcm_hbm, base_hbm, lcm_v, ra_v, o_v, sem):
        wid = lax.axis_index("s") * info.num_cores + lax.axis_index("c")
        base = pl.multiple_of(wid * cols, cols)
        copies = [pltpu.make_async_copy(lcm_hbm.at[pl.ds(0, _LSV)], lcm_v, sem)]
        for k in range(_LSV):
            copies.append(pltpu.make_async_copy(
                ra_hbm.at[pl.ds(k * _N + base, cols)], ra_v.at[k], sem))
        for c in copies:
            c.start()
        for c in copies:
            c.wait()
        sel = lcm_v[...]
        for j in range(cols // 16):
            sl = pl.ds(j * 16, 16)
            acc = sel[0] * ra_v[0, sl]
            for k in range(1, _LSV):
                acc = acc + sel[k] * ra_v[k, sl]
            o_v[sl] = acc
        pltpu.sync_copy(o_v, base_hbm.at[pl.ds(base, cols)])

    return body(ra_flat, lcm_flat)


def _body(x_ref, base_ref, o_ref):
    i = pl.program_id(0)

    @pl.when(i >= _G)
    def _add():
        o_ref[...] = x_ref[...] + base_ref[...]


def kernel(x, running_averages, linear_comb_matrix):
    base = _sc_base(running_averages.reshape(-1), linear_comb_matrix.reshape(-1))
    x2d = x.reshape(_ROWS, _N)
    return pl.pallas_call(
        _body,
        grid=(2 * _G,),
        in_specs=[
            pl.BlockSpec((_R, _N), lambda i: (jnp.where(i < _G, i, i - _G), 0)),
            pl.BlockSpec((1, _N), lambda i: (0, 0)),
        ],
        out_specs=pl.BlockSpec((_R, _N), lambda i: (jnp.maximum(i - _G, 0), 0)),
        out_shape=jax.ShapeDtypeStruct((_ROWS, _N), jnp.float32),
        compiler_params=pltpu.CompilerParams(dimension_semantics=("arbitrary",)),
    )(x2d, base.reshape(1, _N)).reshape(x.shape)


# speedup vs baseline: 1.0164x; 1.0164x over previous
import functools

import jax
import jax.numpy as jnp
from jax import lax
from jax.experimental import pallas as pl
from jax.experimental.pallas import tpu as pltpu
from jax.experimental.pallas import tpu_sc as plsc

_LSV_DATASET_NUM = 16
_N_EMBD = 2048
_EMA_ALPHA = 1.526e-05
_LSV_INDEX = 0
_LSV_SCALING_FACTOR = 1.0

_ROWS = 4 * 8192
_R = 512
_G = _ROWS // _R
_C = 9


def _sc_base(ra_flat, lcm_flat):
    info = plsc.get_sparse_core_info()
    nw = info.num_cores * info.num_subcores
    cols = _N_EMBD // nw
    mesh = plsc.VectorSubcoreMesh(core_axis_name="c", subcore_axis_name="s")

    @functools.partial(
        pl.kernel,
        mesh=mesh,
        out_type=[
            jax.ShapeDtypeStruct((_N_EMBD,), jnp.float32),
            jax.ShapeDtypeStruct((16,), jnp.float32),
        ],
        scratch_types=[
            pltpu.VMEM((_LSV_DATASET_NUM,), jnp.float32),
            pltpu.VMEM((_LSV_DATASET_NUM, cols), jnp.float32),
            pltpu.VMEM((cols,), jnp.float32),
            pltpu.VMEM((16,), jnp.float32),
            pltpu.SemaphoreType.DMA,
        ],
    )
    def body(ra_hbm, lcm_hbm, base_hbm, g_hbm, lcm_v, ra_v, o_v, g_v, sem):
        wid = lax.axis_index("s") * info.num_cores + lax.axis_index("c")
        base = pl.multiple_of(wid * cols, cols)
        copies = [pltpu.make_async_copy(
            lcm_hbm.at[pl.ds(_LSV_INDEX * _LSV_DATASET_NUM, _LSV_DATASET_NUM)],
            lcm_v, sem)]
        for k in range(_LSV_DATASET_NUM):
            copies.append(pltpu.make_async_copy(
                ra_hbm.at[pl.ds(k * _N_EMBD + base, cols)], ra_v.at[k], sem))
        for c in copies:
            c.start()
        for c in copies:
            c.wait()
        sel = lcm_v[...] * _LSV_SCALING_FACTOR
        for j in range(cols // 16):
            sl = pl.ds(j * 16, 16)
            acc = (sel[_LSV_INDEX] * (1.0 - _EMA_ALPHA)) * ra_v[_LSV_INDEX, sl]
            for k in range(_LSV_DATASET_NUM):
                if k == _LSV_INDEX:
                    continue
                acc = acc + sel[k] * ra_v[k, sl]
            o_v[sl] = acc
        pltpu.sync_copy(o_v, base_hbm.at[pl.ds(base, cols)])

        @pl.when(wid == 0)
        def _gamma():
            g_v[...] = sel * (_EMA_ALPHA / float(_ROWS))
            pltpu.sync_copy(g_v, g_hbm)

    return body(ra_flat, lcm_flat)


def _lead_body(x_ref, o_ref, acc_ref):
    i = pl.program_id(0)

    @pl.when(i == 0)
    def _init():
        acc_ref[...] = jnp.zeros_like(acc_ref)

    acc_ref[...] += jnp.sum(x_ref[...].reshape(-1, 8, _N_EMBD), axis=0)

    @pl.when(i == pl.num_programs(0) - 1)
    def _fini():
        o_ref[...] = jnp.sum(acc_ref[...], axis=0, keepdims=True)


def _lead_sums(x2d):
    return pl.pallas_call(
        _lead_body,
        grid=(_G - _C,),
        in_specs=[pl.BlockSpec((_R, _N_EMBD), lambda i: (i + _C, 0))],
        out_specs=pl.BlockSpec((1, _N_EMBD), lambda i: (0, 0)),
        out_shape=jax.ShapeDtypeStruct((1, _N_EMBD), jnp.float32),
        scratch_shapes=[pltpu.VMEM((8, _N_EMBD), jnp.float32)],
        compiler_params=pltpu.CompilerParams(
            dimension_semantics=("arbitrary",)),
    )(x2d)


def _fused_body(x_ref, base_ref, ls_ref, g_ref, out_ref,
                acc_ref, v_ref, cache_ref):
    i = pl.program_id(0)

    @pl.when(i == 0)
    def _init():
        acc_ref[...] = jnp.zeros_like(acc_ref)

    @pl.when(i < _C)
    def _reduce():
        blk = x_ref[...]
        acc_ref[...] += jnp.sum(blk.reshape(-1, 8, _N_EMBD), axis=0)
        cache_ref[pl.ds(i * _R, _R), :] = blk

    @pl.when(i == _C - 1)
    def _combine():
        sums = ls_ref[...] + jnp.sum(acc_ref[...], axis=0, keepdims=True)
        v_ref[...] = base_ref[...] + g_ref[_LSV_INDEX] * sums

    @pl.when(i >= _C)
    def _add():
        j = i - _C
        v = v_ref[...]

        @pl.when(j < _C)
        def _from_cache():
            out_ref[...] = cache_ref[pl.ds(j * _R, _R), :] + v

        @pl.when(j >= _C)
        def _from_hbm():
            out_ref[...] = x_ref[...] + v


def _x_index(i):
    return (jnp.where(i < _C, i, jnp.maximum(i - _C, _C - 1)), 0)


def _fused(x2d, base, lead_sums, gvec):
    return pl.pallas_call(
        _fused_body,
        grid=(_C + _G,),
        in_specs=[
            pl.BlockSpec((_R, _N_EMBD), _x_index),
            pl.BlockSpec((1, _N_EMBD), lambda i: (0, 0)),
            pl.BlockSpec((1, _N_EMBD), lambda i: (0, 0)),
            pl.BlockSpec(memory_space=pltpu.SMEM),
        ],
        out_specs=pl.BlockSpec(
            (_R, _N_EMBD), lambda i: (jnp.maximum(i - _C, 0), 0)),
        out_shape=jax.ShapeDtypeStruct((_ROWS, _N_EMBD), jnp.float32),
        scratch_shapes=[
            pltpu.VMEM((8, _N_EMBD), jnp.float32),
            pltpu.VMEM((1, _N_EMBD), jnp.float32),
            pltpu.VMEM((_C * _R, _N_EMBD), jnp.float32),
        ],
        compiler_params=pltpu.CompilerParams(
            dimension_semantics=("arbitrary",)),
    )(x2d, base, lead_sums, gvec)


def kernel(x, running_averages, linear_comb_matrix):
    base, gvec = _sc_base(
        running_averages.reshape(-1), linear_comb_matrix.reshape(-1))
    x2d = x.reshape(_ROWS, _N_EMBD)
    lead = _lead_sums(x2d)
    out = _fused(x2d, base.reshape(1, _N_EMBD), lead, gvec)
    return out.reshape(x.shape)


# speedup vs baseline: 1.0253x; 1.0087x over previous
import functools

import jax
import jax.numpy as jnp
from jax import lax
from jax.experimental import pallas as pl
from jax.experimental.pallas import tpu as pltpu
from jax.experimental.pallas import tpu_sc as plsc

_LSV_DATASET_NUM = 16
_N_EMBD = 2048
_EMA_ALPHA = 1.526e-05
_LSV_INDEX = 0
_LSV_SCALING_FACTOR = 1.0

_ROWS = 4 * 8192
_R = 512
_G = _ROWS // _R
_C = 8
_RL = 2048
_GL = (_ROWS - _C * _R) // _RL


def _sc_base(ra_flat, lcm_flat):
    info = plsc.get_sparse_core_info()
    nw = info.num_cores * info.num_subcores
    cols = _N_EMBD // nw
    mesh = plsc.VectorSubcoreMesh(core_axis_name="c", subcore_axis_name="s")

    @functools.partial(
        pl.kernel,
        mesh=mesh,
        out_type=[
            jax.ShapeDtypeStruct((_N_EMBD,), jnp.float32),
            jax.ShapeDtypeStruct((16,), jnp.float32),
        ],
        scratch_types=[
            pltpu.VMEM((_LSV_DATASET_NUM,), jnp.float32),
            pltpu.VMEM((_LSV_DATASET_NUM, cols), jnp.float32),
            pltpu.VMEM((cols,), jnp.float32),
            pltpu.VMEM((16,), jnp.float32),
            pltpu.SemaphoreType.DMA,
        ],
    )
    def body(ra_hbm, lcm_hbm, base_hbm, g_hbm, lcm_v, ra_v, o_v, g_v, sem):
        wid = lax.axis_index("s") * info.num_cores + lax.axis_index("c")
        base = pl.multiple_of(wid * cols, cols)
        copies = [pltpu.make_async_copy(
            lcm_hbm.at[pl.ds(_LSV_INDEX * _LSV_DATASET_NUM, _LSV_DATASET_NUM)],
            lcm_v, sem)]
        for k in range(_LSV_DATASET_NUM):
            copies.append(pltpu.make_async_copy(
                ra_hbm.at[pl.ds(k * _N_EMBD + base, cols)], ra_v.at[k], sem))
        for c in copies:
            c.start()
        for c in copies:
            c.wait()
        sel = lcm_v[...] * _LSV_SCALING_FACTOR
        for j in range(cols // 16):
            sl = pl.ds(j * 16, 16)
            acc = (sel[_LSV_INDEX] * (1.0 - _EMA_ALPHA)) * ra_v[_LSV_INDEX, sl]
            for k in range(_LSV_DATASET_NUM):
                if k == _LSV_INDEX:
                    continue
                acc = acc + sel[k] * ra_v[k, sl]
            o_v[sl] = acc
        pltpu.sync_copy(o_v, base_hbm.at[pl.ds(base, cols)])

        @pl.when(wid == 0)
        def _gamma():
            g_v[...] = sel * (_EMA_ALPHA / float(_ROWS))
            pltpu.sync_copy(g_v, g_hbm)

    return body(ra_flat, lcm_flat)


def _lead_body(x_ref, o_ref, acc_ref):
    i = pl.program_id(0)

    @pl.when(i == 0)
    def _init():
        acc_ref[...] = jnp.zeros_like(acc_ref)

    acc_ref[...] += jnp.sum(x_ref[...].reshape(-1, 8, _N_EMBD), axis=0)

    @pl.when(i == pl.num_programs(0) - 1)
    def _fini():
        o_ref[...] = jnp.sum(acc_ref[...], axis=0, keepdims=True)


def _lead_sums(x2d):
    return pl.pallas_call(
        _lead_body,
        grid=(_GL,),
        in_specs=[pl.BlockSpec(
            (_RL, _N_EMBD), lambda i: (i + (_C * _R) // _RL, 0))],
        out_specs=pl.BlockSpec((1, _N_EMBD), lambda i: (0, 0)),
        out_shape=jax.ShapeDtypeStruct((1, _N_EMBD), jnp.float32),
        scratch_shapes=[pltpu.VMEM((8, _N_EMBD), jnp.float32)],
        compiler_params=pltpu.CompilerParams(
            dimension_semantics=("arbitrary",)),
    )(x2d)


def _fused_body(x_ref, base_ref, ls_ref, g_ref, out_ref,
                acc_ref, v_ref, cache_ref):
    i = pl.program_id(0)

    @pl.when(i == 0)
    def _init():
        acc_ref[...] = jnp.zeros_like(acc_ref)

    @pl.when(i < _C)
    def _reduce():
        blk = x_ref[...]
        acc_ref[...] += jnp.sum(blk.reshape(-1, 8, _N_EMBD), axis=0)
        cache_ref[pl.ds(i * _R, _R), :] = blk

    @pl.when(i == _C - 1)
    def _combine():
        sums = ls_ref[...] + jnp.sum(acc_ref[...], axis=0, keepdims=True)
        v_ref[...] = base_ref[...] + g_ref[_LSV_INDEX] * sums

    @pl.when(i >= _C)
    def _add():
        j = i - _C
        v = v_ref[...]

        @pl.when(j < _C)
        def _from_cache():
            out_ref[...] = cache_ref[pl.ds(j * _R, _R), :] + v

        @pl.when(j >= _C)
        def _from_hbm():
            out_ref[...] = x_ref[...] + v


def _x_index(i):
    return (jnp.where(i < _C, i, jnp.maximum(i - _C, _C - 1)), 0)


def _fused(x2d, base, lead_sums, gvec):
    return pl.pallas_call(
        _fused_body,
        grid=(_C + _G,),
        in_specs=[
            pl.BlockSpec((_R, _N_EMBD), _x_index),
            pl.BlockSpec((1, _N_EMBD), lambda i: (0, 0)),
            pl.BlockSpec((1, _N_EMBD), lambda i: (0, 0)),
            pl.BlockSpec(memory_space=pltpu.SMEM),
        ],
        out_specs=pl.BlockSpec(
            (_R, _N_EMBD), lambda i: (jnp.maximum(i - _C, 0), 0)),
        out_shape=jax.ShapeDtypeStruct((_ROWS, _N_EMBD), jnp.float32),
        scratch_shapes=[
            pltpu.VMEM((8, _N_EMBD), jnp.float32),
            pltpu.VMEM((1, _N_EMBD), jnp.float32),
            pltpu.VMEM((_C * _R, _N_EMBD), jnp.float32),
        ],
        compiler_params=pltpu.CompilerParams(
            dimension_semantics=("arbitrary",)),
    )(x2d, base, lead_sums, gvec)


def kernel(x, running_averages, linear_comb_matrix):
    base, gvec = _sc_base(
        running_averages.reshape(-1), linear_comb_matrix.reshape(-1))
    x2d = x.reshape(_ROWS, _N_EMBD)
    lead = _lead_sums(x2d)
    out = _fused(x2d, base.reshape(1, _N_EMBD), lead, gvec)
    return out.reshape(x.shape)


# speedup vs baseline: 1.0305x; 1.0050x over previous
import functools

import jax
import jax.numpy as jnp
from jax import lax
from jax.experimental import pallas as pl
from jax.experimental.pallas import tpu as pltpu
from jax.experimental.pallas import tpu_sc as plsc

_LSV_DATASET_NUM = 16
_N_EMBD = 2048
_EMA_ALPHA = 1.526e-05
_LSV_INDEX = 0
_LSV_SCALING_FACTOR = 1.0

_ROWS = 4 * 8192
_R = 512
_G = _ROWS // _R
_C = 9
_RL = 2560
_GL = (_ROWS - _C * _R) // _RL


def _sc_base(ra_flat, lcm_flat):
    info = plsc.get_sparse_core_info()
    nw = info.num_cores * info.num_subcores
    cols = _N_EMBD // nw
    mesh = plsc.VectorSubcoreMesh(core_axis_name="c", subcore_axis_name="s")

    @functools.partial(
        pl.kernel,
        mesh=mesh,
        out_type=[
            jax.ShapeDtypeStruct((_N_EMBD,), jnp.float32),
            jax.ShapeDtypeStruct((16,), jnp.float32),
        ],
        scratch_types=[
            pltpu.VMEM((_LSV_DATASET_NUM,), jnp.float32),
            pltpu.VMEM((_LSV_DATASET_NUM, cols), jnp.float32),
            pltpu.VMEM((cols,), jnp.float32),
            pltpu.VMEM((16,), jnp.float32),
            pltpu.SemaphoreType.DMA,
        ],
    )
    def body(ra_hbm, lcm_hbm, base_hbm, g_hbm, lcm_v, ra_v, o_v, g_v, sem):
        wid = lax.axis_index("s") * info.num_cores + lax.axis_index("c")
        base = pl.multiple_of(wid * cols, cols)
        copies = [pltpu.make_async_copy(
            lcm_hbm.at[pl.ds(_LSV_INDEX * _LSV_DATASET_NUM, _LSV_DATASET_NUM)],
            lcm_v, sem)]
        for k in range(_LSV_DATASET_NUM):
            copies.append(pltpu.make_async_copy(
                ra_hbm.at[pl.ds(k * _N_EMBD + base, cols)], ra_v.at[k], sem))
        for c in copies:
            c.start()
        for c in copies:
            c.wait()
        sel = lcm_v[...] * _LSV_SCALING_FACTOR
        for j in range(cols // 16):
            sl = pl.ds(j * 16, 16)
            acc = (sel[_LSV_INDEX] * (1.0 - _EMA_ALPHA)) * ra_v[_LSV_INDEX, sl]
            for k in range(_LSV_DATASET_NUM):
                if k == _LSV_INDEX:
                    continue
                acc = acc + sel[k] * ra_v[k, sl]
            o_v[sl] = acc
        pltpu.sync_copy(o_v, base_hbm.at[pl.ds(base, cols)])

        @pl.when(wid == 0)
        def _gamma():
            g_v[...] = sel * (_EMA_ALPHA / float(_ROWS))
            pltpu.sync_copy(g_v, g_hbm)

    return body(ra_flat, lcm_flat)


def _lead_body(x_ref, o_ref, acc_ref):
    i = pl.program_id(0)

    @pl.when(i == 0)
    def _init():
        acc_ref[...] = jnp.zeros_like(acc_ref)

    acc_ref[...] += jnp.sum(x_ref[...].reshape(-1, 8, _N_EMBD), axis=0)

    @pl.when(i == pl.num_programs(0) - 1)
    def _fini():
        o_ref[...] = jnp.sum(acc_ref[...], axis=0, keepdims=True)


def _lead_sums(x2d):
    return pl.pallas_call(
        _lead_body,
        grid=(_GL,),
        in_specs=[pl.BlockSpec((_RL, _N_EMBD), lambda i: (i, 0))],
        out_specs=pl.BlockSpec((1, _N_EMBD), lambda i: (0, 0)),
        out_shape=jax.ShapeDtypeStruct((1, _N_EMBD), jnp.float32),
        scratch_shapes=[pltpu.VMEM((8, _N_EMBD), jnp.float32)],
        compiler_params=pltpu.CompilerParams(
            dimension_semantics=("arbitrary",)),
    )(x2d)


def _fused_body(x_ref, base_ref, ls_ref, g_ref, out_ref,
                acc_ref, v_ref, cache_ref):
    i = pl.program_id(0)

    @pl.when(i == 0)
    def _init():
        acc_ref[...] = jnp.zeros_like(acc_ref)

    @pl.when(i < _C)
    def _reduce():
        blk = x_ref[...]
        acc_ref[...] += jnp.sum(blk.reshape(-1, 8, _N_EMBD), axis=0)
        cache_ref[pl.ds(i * _R, _R), :] = blk

    @pl.when(i == _C - 1)
    def _combine():
        sums = ls_ref[...] + jnp.sum(acc_ref[...], axis=0, keepdims=True)
        v_ref[...] = base_ref[...] + g_ref[_LSV_INDEX] * sums

    @pl.when(i >= _C)
    def _add():
        j = i - _C
        v = v_ref[...]

        @pl.when(j < _G - _C)
        def _from_hbm():
            out_ref[...] = x_ref[...] + v

        @pl.when(j >= _G - _C)
        def _from_cache():
            out_ref[...] = cache_ref[pl.ds((j - (_G - _C)) * _R, _R), :] + v


def _x_index(i):
    return (jnp.where(i < _C, (_G - _C) + i,
                      jnp.minimum(i - _C, _G - _C - 1)), 0)


def _fused(x2d, base, lead_sums, gvec):
    return pl.pallas_call(
        _fused_body,
        grid=(_C + _G,),
        in_specs=[
            pl.BlockSpec((_R, _N_EMBD), _x_index),
            pl.BlockSpec((1, _N_EMBD), lambda i: (0, 0)),
            pl.BlockSpec((1, _N_EMBD), lambda i: (0, 0)),
            pl.BlockSpec(memory_space=pltpu.SMEM),
        ],
        out_specs=pl.BlockSpec(
            (_R, _N_EMBD), lambda i: (jnp.maximum(i - _C, 0), 0)),
        out_shape=jax.ShapeDtypeStruct((_ROWS, _N_EMBD), jnp.float32),
        scratch_shapes=[
            pltpu.VMEM((8, _N_EMBD), jnp.float32),
            pltpu.VMEM((1, _N_EMBD), jnp.float32),
            pltpu.VMEM((_C * _R, _N_EMBD), jnp.float32),
        ],
        compiler_params=pltpu.CompilerParams(
            dimension_semantics=("arbitrary",)),
    )(x2d, base, lead_sums, gvec)


def kernel(x, running_averages, linear_comb_matrix):
    base, gvec = _sc_base(
        running_averages.reshape(-1), linear_comb_matrix.reshape(-1))
    x2d = x.reshape(_ROWS, _N_EMBD)
    lead = _lead_sums(x2d)
    out = _fused(x2d, base.reshape(1, _N_EMBD), lead, gvec)
    return out.reshape(x.shape)
